# Initial kernel scaffold; baseline (speedup 1.0000x reference)
import functools
import math

import jax
import jax.numpy as jnp
from jax import lax
from jax.experimental import pallas as pl
from jax.experimental.pallas import tpu as pltpu
from jax.experimental.pallas import tpu_sc as plsc

MODEL_DIM = 128
SCALE = math.sqrt(MODEL_DIM)

_INFO = plsc.get_sparse_core_info()
NUM_CORES = _INFO.num_cores
NUM_SUBCORES = _INFO.num_subcores
NUM_WORKERS = NUM_CORES * NUM_SUBCORES

SUB = 128
K = 4


def _scale_body(t_ref, o_ref):
    o_ref[...] = t_ref[...] * SCALE


def _scale_table(table):
    vocab, d = table.shape
    block = 2000
    assert vocab % block == 0
    return pl.pallas_call(
        _scale_body,
        grid=(vocab // block,),
        in_specs=[pl.BlockSpec((block, d), lambda i: (i, 0))],
        out_specs=pl.BlockSpec((block, d), lambda i: (i, 0)),
        out_shape=jax.ShapeDtypeStruct((vocab, d), table.dtype),
    )(table)


def _gather_kernel(n_total, d):
    per_worker = n_total // NUM_WORKERS
    rows_per_round = K * SUB
    n_rounds = per_worker // rows_per_round
    assert per_worker % rows_per_round == 0

    mesh = plsc.VectorSubcoreMesh(core_axis_name="c", subcore_axis_name="s")

    @functools.partial(
        pl.kernel,
        mesh=mesh,
        out_type=jax.ShapeDtypeStruct((n_total, d), jnp.float32),
        scratch_types=[
            pltpu.VMEM((K, SUB), jnp.int32),
            pltpu.VMEM((K * SUB, d), jnp.float32),
            pltpu.SemaphoreType.DMA,
        ],
    )
    def gather(table_hbm, idx_hbm, out_hbm, idx_v, rows_v, sem):
        wid = lax.axis_index("s") * NUM_CORES + lax.axis_index("c")
        base = wid * per_worker

        def round_body(r, carry):
            start = base + r * rows_per_round
            pltpu.sync_copy(
                idx_hbm.at[pl.ds(start, rows_per_round)].reshape(K, SUB),
                idx_v,
            )
            copies = []
            for j in range(K):
                copies.append(
                    pltpu.async_copy(
                        table_hbm.at[idx_v.at[j]],
                        rows_v.at[pl.ds(j * SUB, SUB)],
                        sem,
                    )
                )
            for c in copies:
                c.wait()
            pltpu.sync_copy(rows_v, out_hbm.at[pl.ds(start, rows_per_round)])
            return carry

        lax.fori_loop(0, n_rounds, round_body, 0)

    return gather


def kernel(token_ids_batch, embeddings_table):
    batch, seq = token_ids_batch.shape
    vocab, d = embeddings_table.shape
    n_total = batch * seq

    scaled = _scale_table(embeddings_table)
    flat_ids = token_ids_batch.reshape(n_total).astype(jnp.int32)
    out = _gather_kernel(n_total, d)(scaled, flat_ids)
    return out.reshape(batch, seq, d)


# speedup vs baseline: 7.1467x; 7.1467x over previous
import functools
import math

import jax
import jax.numpy as jnp
from jax import lax
from jax.experimental import pallas as pl
from jax.experimental.pallas import tpu as pltpu
from jax.experimental.pallas import tpu_sc as plsc

MODEL_DIM = 128
SCALE = math.sqrt(MODEL_DIM)

_INFO = plsc.get_sparse_core_info()
NUM_CORES = _INFO.num_cores
NUM_SUBCORES = _INFO.num_subcores
NUM_WORKERS = NUM_CORES * NUM_SUBCORES

SUB = 128
K = 4


def _scale_body(t_ref, o_ref):
    o_ref[...] = t_ref[...] * SCALE


def _scale_table(table):
    vocab, d = table.shape
    block = 2000
    assert vocab % block == 0
    return pl.pallas_call(
        _scale_body,
        grid=(vocab // block,),
        in_specs=[pl.BlockSpec((block, d), lambda i: (i, 0))],
        out_specs=pl.BlockSpec((block, d), lambda i: (i, 0)),
        out_shape=jax.ShapeDtypeStruct((vocab, d), table.dtype),
    )(table)


def _gather_kernel(n_total, d):
    per_worker = n_total // NUM_WORKERS
    rows_per_round = K * SUB
    n_rounds = per_worker // rows_per_round
    assert per_worker % rows_per_round == 0

    mesh = plsc.VectorSubcoreMesh(core_axis_name="c", subcore_axis_name="s")

    @functools.partial(
        pl.kernel,
        mesh=mesh,
        out_type=jax.ShapeDtypeStruct((n_total, d), jnp.float32),
        scratch_types=[
            pltpu.VMEM((K * SUB,), jnp.int32),
            pltpu.VMEM((K * SUB, d), jnp.float32),
            pltpu.SemaphoreType.DMA,
        ],
    )
    def gather(table_hbm, idx_hbm, out_hbm, idx_v, rows_v, sem):
        wid = lax.axis_index("s") * NUM_CORES + lax.axis_index("c")
        base = wid * per_worker

        def round_body(r, carry):
            start = base + r * rows_per_round
            pltpu.sync_copy(idx_hbm.at[pl.ds(start, rows_per_round)], idx_v)
            copies = []
            for j in range(K):
                copies.append(
                    pltpu.async_copy(
                        table_hbm.at[idx_v.at[pl.ds(j * SUB, SUB)]],
                        rows_v.at[pl.ds(j * SUB, SUB)],
                        sem,
                    )
                )
            for c in copies:
                c.wait()
            pltpu.sync_copy(rows_v, out_hbm.at[pl.ds(start, rows_per_round)])
            return carry

        lax.fori_loop(0, n_rounds, round_body, 0)

    return gather


def kernel(token_ids_batch, embeddings_table):
    batch, seq = token_ids_batch.shape
    vocab, d = embeddings_table.shape
    n_total = batch * seq

    scaled = _scale_table(embeddings_table)
    flat_ids = token_ids_batch.reshape(n_total).astype(jnp.int32)
    out = _gather_kernel(n_total, d)(scaled, flat_ids)
    return out.reshape(batch, seq, d)


# speedup vs baseline: 7.9370x; 1.1106x over previous
import functools
import math

import jax
import jax.numpy as jnp
from jax import lax
from jax.experimental import pallas as pl
from jax.experimental.pallas import tpu as pltpu
from jax.experimental.pallas import tpu_sc as plsc

MODEL_DIM = 128
SCALE = math.sqrt(MODEL_DIM)

_INFO = plsc.get_sparse_core_info()
NUM_CORES = _INFO.num_cores
NUM_SUBCORES = _INFO.num_subcores
NUM_WORKERS = NUM_CORES * NUM_SUBCORES

SUB = 128
K = 2


def _scale_body(t_ref, o_ref):
    o_ref[...] = t_ref[...] * SCALE


def _scale_table(table):
    vocab, d = table.shape
    block = 2000
    assert vocab % block == 0
    return pl.pallas_call(
        _scale_body,
        grid=(vocab // block,),
        in_specs=[pl.BlockSpec((block, d), lambda i: (i, 0))],
        out_specs=pl.BlockSpec((block, d), lambda i: (i, 0)),
        out_shape=jax.ShapeDtypeStruct((vocab, d), table.dtype),
    )(table)


def _gather_kernel(n_total, d):
    per_worker = n_total // NUM_WORKERS
    rows_per_round = K * SUB
    n_rounds = per_worker // rows_per_round
    assert per_worker % rows_per_round == 0

    mesh = plsc.VectorSubcoreMesh(core_axis_name="c", subcore_axis_name="s")

    R = rows_per_round
    n_half = n_rounds // 2
    assert n_rounds % 2 == 0

    @functools.partial(
        pl.kernel,
        mesh=mesh,
        out_type=jax.ShapeDtypeStruct((n_total, d), jnp.float32),
        scratch_types=[
            pltpu.VMEM((per_worker,), jnp.int32),
            pltpu.VMEM((R, d), jnp.float32),
            pltpu.VMEM((R, d), jnp.float32),
            pltpu.SemaphoreType.DMA,
            pltpu.SemaphoreType.DMA,
            pltpu.SemaphoreType.DMA,
            pltpu.SemaphoreType.DMA,
        ],
    )
    def gather(
        table_hbm, idx_hbm, out_hbm, idx_v, rows_a, rows_b, gsa, gsb, wsa, wsb
    ):
        wid = lax.axis_index("s") * NUM_CORES + lax.axis_index("c")
        base = wid * per_worker

        pltpu.sync_copy(idx_hbm.at[pl.ds(base, per_worker)], idx_v)

        def fire_gather(r, rows_buf, sem):
            for j in range(K):
                pltpu.async_copy(
                    table_hbm.at[idx_v.at[pl.ds(r * R + j * SUB, SUB)]],
                    rows_buf.at[pl.ds(j * SUB, SUB)],
                    sem,
                )

        def wait_gather(rows_buf, sem):
            pltpu.make_async_copy(
                table_hbm.at[pl.ds(0, R)], rows_buf, sem
            ).wait()

        def fire_wb(r, rows_buf, sem):
            pltpu.async_copy(rows_buf, out_hbm.at[pl.ds(base + r * R, R)], sem)

        def wait_wb(rows_buf, sem):
            pltpu.make_async_copy(
                rows_buf, out_hbm.at[pl.ds(base, R)], sem
            ).wait()

        fire_gather(0, rows_a, gsa)

        def body(i, carry):
            ra = 2 * i
            rb = 2 * i + 1

            @pl.when(i > 0)
            def _():
                wait_wb(rows_b, wsb)

            fire_gather(rb, rows_b, gsb)
            wait_gather(rows_a, gsa)
            fire_wb(ra, rows_a, wsa)
            wait_wb(rows_a, wsa)

            @pl.when(i + 1 < n_half)
            def _():
                fire_gather(ra + 2, rows_a, gsa)

            wait_gather(rows_b, gsb)
            fire_wb(rb, rows_b, wsb)
            return carry

        lax.fori_loop(0, n_half, body, 0)
        wait_wb(rows_b, wsb)

    return gather


def kernel(token_ids_batch, embeddings_table):
    batch, seq = token_ids_batch.shape
    vocab, d = embeddings_table.shape
    n_total = batch * seq

    scaled = _scale_table(embeddings_table)
    flat_ids = token_ids_batch.reshape(n_total).astype(jnp.int32)
    out = _gather_kernel(n_total, d)(scaled, flat_ids)
    return out.reshape(batch, seq, d)


# speedup vs baseline: 7.9469x; 1.0012x over previous
import functools
import math

import jax
import jax.numpy as jnp
from jax import lax
from jax.experimental import pallas as pl
from jax.experimental.pallas import tpu as pltpu
from jax.experimental.pallas import tpu_sc as plsc

MODEL_DIM = 128
SCALE = math.sqrt(MODEL_DIM)

_INFO = plsc.get_sparse_core_info()
NUM_CORES = _INFO.num_cores
NUM_SUBCORES = _INFO.num_subcores
NUM_WORKERS = NUM_CORES * NUM_SUBCORES

SUB = 128


def _scale_body(t_ref, o_ref):
    o_ref[...] = t_ref[...] * SCALE


def _scale_table(table):
    vocab, d = table.shape
    block = 2000
    assert vocab % block == 0
    return pl.pallas_call(
        _scale_body,
        grid=(vocab // block,),
        in_specs=[pl.BlockSpec((block, d), lambda i: (i, 0))],
        out_specs=pl.BlockSpec((block, d), lambda i: (i, 0)),
        out_shape=jax.ShapeDtypeStruct((vocab, d), table.dtype),
    )(table)


def _gather_kernel(n_total, d):
    per_worker = n_total // NUM_WORKERS

    mesh = plsc.VectorSubcoreMesh(core_axis_name="c", subcore_axis_name="s")

    R = SUB
    NBUF = 4
    n_rounds = per_worker // R
    n_iters = n_rounds // NBUF
    assert n_rounds % NBUF == 0

    @functools.partial(
        pl.kernel,
        mesh=mesh,
        out_type=jax.ShapeDtypeStruct((n_total, d), jnp.float32),
        scratch_types=[
            pltpu.VMEM((per_worker,), jnp.int32),
            pltpu.VMEM((NBUF, R, d), jnp.float32),
            pltpu.SemaphoreType.DMA,
            pltpu.SemaphoreType.DMA,
            pltpu.SemaphoreType.DMA,
            pltpu.SemaphoreType.DMA,
            pltpu.SemaphoreType.DMA,
            pltpu.SemaphoreType.DMA,
            pltpu.SemaphoreType.DMA,
            pltpu.SemaphoreType.DMA,
        ],
    )
    def gather(table_hbm, idx_hbm, out_hbm, idx_v, rows_v, *sems):
        gs = sems[:NBUF]
        ws = sems[NBUF:]
        wid = lax.axis_index("s") * NUM_CORES + lax.axis_index("c")
        base = wid * per_worker

        pltpu.sync_copy(idx_hbm.at[pl.ds(base, per_worker)], idx_v)

        def fire_gather(r, b):
            pltpu.async_copy(
                table_hbm.at[idx_v.at[pl.ds(r * R, R)]], rows_v.at[b], gs[b]
            )

        def wait_gather(b):
            pltpu.make_async_copy(
                table_hbm.at[pl.ds(0, R)], rows_v.at[b], gs[b]
            ).wait()

        def fire_wb(r, b):
            pltpu.async_copy(
                rows_v.at[b], out_hbm.at[pl.ds(base + r * R, R)], ws[b]
            )

        def wait_wb(b):
            pltpu.make_async_copy(
                rows_v.at[b], out_hbm.at[pl.ds(base, R)], ws[b]
            ).wait()

        fire_gather(0, 0)
        fire_gather(1, 1)

        def body(i, carry):
            r0 = NBUF * i

            @pl.when(i > 0)
            def _():
                wait_wb(2)

            fire_gather(r0 + 2, 2)
            wait_gather(0)
            fire_wb(r0, 0)

            @pl.when(i > 0)
            def _():
                wait_wb(3)

            fire_gather(r0 + 3, 3)
            wait_gather(1)
            fire_wb(r0 + 1, 1)

            wait_wb(0)

            @pl.when(i + 1 < n_iters)
            def _():
                fire_gather(r0 + 4, 0)

            wait_gather(2)
            fire_wb(r0 + 2, 2)

            wait_wb(1)

            @pl.when(i + 1 < n_iters)
            def _():
                fire_gather(r0 + 5, 1)

            wait_gather(3)
            fire_wb(r0 + 3, 3)
            return carry

        lax.fori_loop(0, n_iters, body, 0)
        wait_wb(2)
        wait_wb(3)

    return gather


def kernel(token_ids_batch, embeddings_table):
    batch, seq = token_ids_batch.shape
    vocab, d = embeddings_table.shape
    n_total = batch * seq

    scaled = _scale_table(embeddings_table)
    flat_ids = token_ids_batch.reshape(n_total).astype(jnp.int32)
    out = _gather_kernel(n_total, d)(scaled, flat_ids)
    return out.reshape(batch, seq, d)


# speedup vs baseline: 9.1402x; 1.1501x over previous
import functools
import math

import jax
import jax.numpy as jnp
from jax import lax
from jax.experimental import pallas as pl
from jax.experimental.pallas import tpu as pltpu
from jax.experimental.pallas import tpu_sc as plsc

MODEL_DIM = 128
SCALE = math.sqrt(MODEL_DIM)

_INFO = plsc.get_sparse_core_info()
NUM_CORES = _INFO.num_cores
NUM_SUBCORES = _INFO.num_subcores
NUM_WORKERS = NUM_CORES * NUM_SUBCORES

SUB = 128


def _scale_body(t_ref, o_ref):
    o_ref[...] = t_ref[...] * SCALE


def _scale_table(table):
    vocab, d = table.shape
    block = 2000
    assert vocab % block == 0
    return pl.pallas_call(
        _scale_body,
        grid=(vocab // block,),
        in_specs=[pl.BlockSpec((block, d), lambda i: (i, 0))],
        out_specs=pl.BlockSpec((block, d), lambda i: (i, 0)),
        out_shape=jax.ShapeDtypeStruct((vocab, d), table.dtype),
    )(table)


def _gather_kernel(n_total, d):
    per_worker = n_total // NUM_WORKERS

    mesh = plsc.VectorSubcoreMesh(core_axis_name="c", subcore_axis_name="s")

    R = SUB
    NBUF = 4
    n_rounds = per_worker // R
    n_iters = n_rounds // NBUF
    assert n_rounds % NBUF == 0

    @functools.partial(
        pl.kernel,
        mesh=mesh,
        out_type=jax.ShapeDtypeStruct((n_total, d), jnp.float32),
        scratch_types=[
            pltpu.VMEM((per_worker,), jnp.int32),
            pltpu.VMEM((NBUF, R, d), jnp.float32),
            pltpu.SemaphoreType.DMA,
            pltpu.SemaphoreType.DMA,
            pltpu.SemaphoreType.DMA,
            pltpu.SemaphoreType.DMA,
            pltpu.SemaphoreType.DMA,
            pltpu.SemaphoreType.DMA,
            pltpu.SemaphoreType.DMA,
            pltpu.SemaphoreType.DMA,
        ],
    )
    def gather(table_hbm, idx_hbm, out_hbm, idx_v, rows_v, *sems):
        gs = sems[:NBUF]
        ws = sems[NBUF:]
        wid = lax.axis_index("s") * NUM_CORES + lax.axis_index("c")
        base = wid * per_worker

        pltpu.sync_copy(idx_hbm.at[pl.ds(base, per_worker)], idx_v)

        def fire_gather(r, b):
            pltpu.async_copy(
                table_hbm.at[idx_v.at[pl.ds(r * R, R)]], rows_v.at[b], gs[b]
            )

        def wait_gather(b):
            pltpu.make_async_copy(
                table_hbm.at[pl.ds(0, R)], rows_v.at[b], gs[b]
            ).wait()

        def fire_wb(r, b):
            pltpu.async_copy(
                rows_v.at[b], out_hbm.at[pl.ds(base + r * R, R)], ws[b]
            )

        def wait_wb(b):
            pltpu.make_async_copy(
                rows_v.at[b], out_hbm.at[pl.ds(base, R)], ws[b]
            ).wait()

        def scale_buf(b):
            def sbody(rr, c):
                for j in range(d // 16):
                    v = rows_v[b, rr, pl.ds(j * 16, 16)]
                    rows_v[b, rr, pl.ds(j * 16, 16)] = v * SCALE
                return c

            lax.fori_loop(0, R, sbody, 0)

        fire_gather(0, 0)
        fire_gather(1, 1)

        def body(i, carry):
            r0 = NBUF * i

            @pl.when(i > 0)
            def _():
                wait_wb(2)

            fire_gather(r0 + 2, 2)
            wait_gather(0)
            scale_buf(0)
            fire_wb(r0, 0)

            @pl.when(i > 0)
            def _():
                wait_wb(3)

            fire_gather(r0 + 3, 3)
            wait_gather(1)
            scale_buf(1)
            fire_wb(r0 + 1, 1)

            wait_wb(0)

            @pl.when(i + 1 < n_iters)
            def _():
                fire_gather(r0 + 4, 0)

            wait_gather(2)
            scale_buf(2)
            fire_wb(r0 + 2, 2)

            wait_wb(1)

            @pl.when(i + 1 < n_iters)
            def _():
                fire_gather(r0 + 5, 1)

            wait_gather(3)
            scale_buf(3)
            fire_wb(r0 + 3, 3)
            return carry

        lax.fori_loop(0, n_iters, body, 0)
        wait_wb(2)
        wait_wb(3)

    return gather


def kernel(token_ids_batch, embeddings_table):
    batch, seq = token_ids_batch.shape
    vocab, d = embeddings_table.shape
    n_total = batch * seq

    scaled = embeddings_table
    flat_ids = token_ids_batch.reshape(n_total).astype(jnp.int32)
    out = _gather_kernel(n_total, d)(scaled, flat_ids)
    return out.reshape(batch, seq, d)


# speedup vs baseline: 9.1792x; 1.0043x over previous
import functools
import math

import jax
import jax.numpy as jnp
from jax import lax
from jax.experimental import pallas as pl
from jax.experimental.pallas import tpu as pltpu
from jax.experimental.pallas import tpu_sc as plsc

MODEL_DIM = 128
SCALE = math.sqrt(MODEL_DIM)

_INFO = plsc.get_sparse_core_info()
NUM_CORES = _INFO.num_cores
NUM_SUBCORES = _INFO.num_subcores
NUM_WORKERS = NUM_CORES * NUM_SUBCORES

R = 128
NBUF = 4


def _gather_kernel(n_total, d):
    per_worker = n_total // NUM_WORKERS
    n_rounds = per_worker // R
    n_iters = n_rounds // NBUF
    assert per_worker % R == 0 and n_rounds % NBUF == 0

    mesh = plsc.VectorSubcoreMesh(core_axis_name="c", subcore_axis_name="s")

    @functools.partial(
        pl.kernel,
        mesh=mesh,
        out_type=jax.ShapeDtypeStruct((n_total, d), jnp.float32),
        scratch_types=[
            pltpu.VMEM((per_worker,), jnp.int32),
            pltpu.VMEM((NBUF, R, d), jnp.float32),
            pltpu.SemaphoreType.DMA,
            pltpu.SemaphoreType.DMA,
            pltpu.SemaphoreType.DMA,
            pltpu.SemaphoreType.DMA,
            pltpu.SemaphoreType.DMA,
            pltpu.SemaphoreType.DMA,
            pltpu.SemaphoreType.DMA,
            pltpu.SemaphoreType.DMA,
        ],
    )
    def gather(table_hbm, idx_hbm, out_hbm, idx_v, rows_v, *sems):
        gs = sems[:NBUF]
        ws = sems[NBUF:]
        wid = lax.axis_index("s") * NUM_CORES + lax.axis_index("c")
        base = wid * per_worker

        pltpu.sync_copy(idx_hbm.at[pl.ds(base, per_worker)], idx_v)

        def fire_gather(r, b):
            pltpu.async_copy(
                table_hbm.at[idx_v.at[pl.ds(r * R, R)]], rows_v.at[b], gs[b]
            )

        def wait_gather(b):
            pltpu.make_async_copy(
                table_hbm.at[pl.ds(0, R)], rows_v.at[b], gs[b]
            ).wait()

        def fire_wb(r, b):
            pltpu.async_copy(
                rows_v.at[b], out_hbm.at[pl.ds(base + r * R, R)], ws[b]
            )

        def wait_wb(b):
            pltpu.make_async_copy(
                rows_v.at[b], out_hbm.at[pl.ds(base, R)], ws[b]
            ).wait()

        def scale_buf(b):
            def sbody(rr, c):
                for j in range(d // 16):
                    v = rows_v[b, rr, pl.ds(j * 16, 16)]
                    rows_v[b, rr, pl.ds(j * 16, 16)] = v * SCALE
                return c

            lax.fori_loop(0, R, sbody, 0)

        fire_gather(0, 0)
        fire_gather(1, 1)

        def body(i, carry):
            r0 = NBUF * i

            @pl.when(i > 0)
            def _():
                wait_wb(2)

            fire_gather(r0 + 2, 2)
            wait_gather(0)
            scale_buf(0)
            fire_wb(r0, 0)

            @pl.when(i > 0)
            def _():
                wait_wb(3)

            fire_gather(r0 + 3, 3)
            wait_gather(1)
            scale_buf(1)
            fire_wb(r0 + 1, 1)

            wait_wb(0)

            @pl.when(i + 1 < n_iters)
            def _():
                fire_gather(r0 + 4, 0)

            wait_gather(2)
            scale_buf(2)
            fire_wb(r0 + 2, 2)

            wait_wb(1)

            @pl.when(i + 1 < n_iters)
            def _():
                fire_gather(r0 + 5, 1)

            wait_gather(3)
            scale_buf(3)
            fire_wb(r0 + 3, 3)
            return carry

        lax.fori_loop(0, n_iters, body, 0)
        wait_wb(2)
        wait_wb(3)

    return gather


def kernel(token_ids_batch, embeddings_table):
    batch, seq = token_ids_batch.shape
    _, d = embeddings_table.shape
    n_total = batch * seq

    flat_ids = token_ids_batch.reshape(n_total).astype(jnp.int32)
    out = _gather_kernel(n_total, d)(embeddings_table, flat_ids)
    return out.reshape(batch, seq, d)
